# baseline (device time: 103467 ns/iter reference)
import jax
import jax.numpy as jnp
from jax import lax
from jax.experimental import pallas as pl
from jax.experimental.pallas import tpu as pltpu

N_DEV = 8
MASKS = {"x": 1, "y": 3, "z": 4}

GROUPS = [
    {"start": 0, "rows": 768, "order": "xyz"},
    {"start": 768, "rows": 640, "order": "yzx"},
    {"start": 1408, "rows": 640, "order": "zxy"},
]
NG = len(GROUPS)
F32 = jnp.float32
BF16 = jnp.bfloat16


def _keep_high_bit(dim, my):
    if dim == "x":
        return (my ^ (my >> 1)) & 1
    if dim == "y":
        return (my >> 1) & 1
    return (my >> 2) & 1


def kernel(A, B):
    m, k = A.shape
    _, n = B.shape

    def body(a_ref, b_ref, out_ref, zbuf, bbuf, *scr):
        rbufs = scr[0:NG]
        rs_send, rs_recv, ag_send, ag_recv = scr[NG:NG + 4]
        my = lax.axis_index("i")
        nsteps = 3

        lo = [None] * NG
        length = [g["rows"] for g in GROUPS]
        rs_rdma = [None] * NG
        keep = [None] * NG

        bbuf[...] = b_ref[...].astype(BF16)

        def dot_rows(row_lo, rows):
            zbuf[pl.ds(row_lo, rows), :] = jnp.dot(
                a_ref[pl.ds(row_lo, rows), :].astype(BF16), bbuf[...],
                preferred_element_type=F32,
            ).astype(BF16)

        def rs_issue(g, j, send_lo, buf_off, half, dim):
            rdma = pltpu.make_async_remote_copy(
                src_ref=zbuf.at[pl.ds(send_lo, half), :],
                dst_ref=rbufs[g].at[pl.ds(buf_off, half), :],
                send_sem=rs_send.at[g, j],
                recv_sem=rs_recv.at[g, j],
                device_id=(my ^ MASKS[dim],),
                device_id_type=pl.DeviceIdType.MESH,
            )
            rdma.start()
            rs_rdma[g] = rdma

        def rs_step(g, j):
            rs_rdma[g].wait()
            pk_lo, p_half, p_off = keep[g]
            zbuf[pl.ds(pk_lo, p_half), :] = (
                zbuf[pl.ds(pk_lo, p_half), :]
                + rbufs[g][pl.ds(p_off, p_half), :]
            )
            half = p_half // 2
            dim = GROUPS[g]["order"][j]
            b = _keep_high_bit(dim, my)
            send_lo = pk_lo + (1 - b) * half
            off = GROUPS[g]["rows"] - p_half
            rs_issue(g, j, send_lo, off, half, dim)
            keep[g] = (pk_lo + b * half, half, off)
            lo[g] = pk_lo + b * half
            length[g] = half

        for g, G in enumerate(GROUPS):
            half = G["rows"] // 2
            dim = G["order"][0]
            b = _keep_high_bit(dim, my)
            send_lo = G["start"] + (1 - b) * half
            dot_rows(send_lo, half)
            rs_issue(g, 0, send_lo, 0, half, dim)
            lo[g] = G["start"] + b * half
            keep[g] = (lo[g], half, 0)
            length[g] = half
        for g, G in enumerate(GROUPS):
            keep_lo, half, _ = keep[g]
            dot_rows(keep_lo, half)

        for j in range(1, nsteps):
            for g in range(NG):
                rs_step(g, j)

        ag_rdma = [None] * NG

        def ag_issue(g, j):
            L = length[g]
            rdma = pltpu.make_async_remote_copy(
                src_ref=zbuf.at[pl.ds(lo[g], L), :],
                dst_ref=zbuf.at[pl.ds(lo[g], L), :],
                send_sem=ag_send.at[g, j],
                recv_sem=ag_recv.at[g, j],
                device_id=(my ^ MASKS[GROUPS[g]["order"][2 - j]],),
                device_id_type=pl.DeviceIdType.MESH,
            )
            rdma.start()
            ag_rdma[g] = rdma

        for g in range(NG):
            rs_rdma[g].wait()
            keep_lo, L, off = keep[g]
            red = (
                zbuf[pl.ds(keep_lo, L), :].astype(F32)
                + rbufs[g][pl.ds(off, L), :].astype(F32)
            )
            silu = red / (1.0 + jnp.exp(-red))
            zbuf[pl.ds(keep_lo, L), :] = silu.astype(BF16)
            ag_issue(g, 0)

        def ag_consume(g, j):
            ag_rdma[g].wait()
            b = _keep_high_bit(GROUPS[g]["order"][2 - j], my)
            lo[g] = lo[g] - b * length[g]
            length[g] = 2 * length[g]
            if j + 1 < nsteps:
                ag_issue(g, j + 1)
            else:
                G = GROUPS[g]
                out_ref[pl.ds(G["start"], G["rows"]), :] = (
                    zbuf[pl.ds(G["start"], G["rows"]), :].astype(F32)
                )

        for j in range(nsteps):
            for g in range(NG):
                ag_consume(g, j)

    return pl.pallas_call(
        body,
        out_shape=jax.ShapeDtypeStruct((m, n), F32),
        in_specs=[
            pl.BlockSpec(memory_space=pltpu.VMEM),
            pl.BlockSpec(memory_space=pltpu.VMEM),
        ],
        out_specs=pl.BlockSpec(memory_space=pltpu.VMEM),
        scratch_shapes=[
            pltpu.VMEM((m, n), BF16),
            pltpu.VMEM((k, n), BF16),
            *[pltpu.VMEM((g["rows"] * 7 // 8, n), BF16) for g in GROUPS],
            pltpu.SemaphoreType.DMA((NG, 3)),
            pltpu.SemaphoreType.DMA((NG, 3)),
            pltpu.SemaphoreType.DMA((NG, 3)),
            pltpu.SemaphoreType.DMA((NG, 3)),
        ],
        compiler_params=pltpu.CompilerParams(
            vmem_limit_bytes=100 * 1024 * 1024,
        ),
    )(A, B)


# device time: 91907 ns/iter; 1.1258x vs baseline; 1.1258x over previous
import jax
import jax.numpy as jnp
from jax import lax
from jax.experimental import pallas as pl
from jax.experimental.pallas import tpu as pltpu

N_DEV = 8
MASKS = {"x": 1, "y": 3, "z": 4}

GROUPS = [
    {"start": 0, "rows": 256, "order": "xyz"},
    {"start": 256, "rows": 256, "order": "yzx"},
    {"start": 512, "rows": 256, "order": "zxy"},
    {"start": 768, "rows": 256, "order": "xyz"},
    {"start": 1024, "rows": 256, "order": "yzx"},
    {"start": 1280, "rows": 256, "order": "zxy"},
    {"start": 1536, "rows": 256, "order": "xyz"},
    {"start": 1792, "rows": 256, "order": "yzx"},
]
NG = len(GROUPS)
F32 = jnp.float32
BF16 = jnp.bfloat16


def _keep_high_bit(dim, my):
    if dim == "x":
        return (my ^ (my >> 1)) & 1
    if dim == "y":
        return (my >> 1) & 1
    return (my >> 2) & 1


def kernel(A, B):
    m, k = A.shape
    _, n = B.shape

    def body(a_ref, b_ref, out_ref, zbuf, bbuf, *scr):
        rbufs = scr[0:NG]
        rs_send, rs_recv, ag_send, ag_recv = scr[NG:NG + 4]
        my = lax.axis_index("i")
        nsteps = 3

        lo = [None] * NG
        length = [g["rows"] for g in GROUPS]
        rs_rdma = [None] * NG
        keep = [None] * NG

        bbuf[...] = b_ref[...].astype(BF16)

        def dot_rows(row_lo, rows):
            zbuf[pl.ds(row_lo, rows), :] = jnp.dot(
                a_ref[pl.ds(row_lo, rows), :].astype(BF16), bbuf[...],
                preferred_element_type=F32,
            ).astype(BF16)

        def rs_issue(g, j, send_lo, buf_off, half, dim):
            rdma = pltpu.make_async_remote_copy(
                src_ref=zbuf.at[pl.ds(send_lo, half), :],
                dst_ref=rbufs[g].at[pl.ds(buf_off, half), :],
                send_sem=rs_send.at[g, j],
                recv_sem=rs_recv.at[g, j],
                device_id=(my ^ MASKS[dim],),
                device_id_type=pl.DeviceIdType.MESH,
            )
            rdma.start()
            rs_rdma[g] = rdma

        def rs_step(g, j):
            rs_rdma[g].wait()
            pk_lo, p_half, p_off = keep[g]
            zbuf[pl.ds(pk_lo, p_half), :] = (
                zbuf[pl.ds(pk_lo, p_half), :]
                + rbufs[g][pl.ds(p_off, p_half), :]
            )
            half = p_half // 2
            dim = GROUPS[g]["order"][j]
            b = _keep_high_bit(dim, my)
            send_lo = pk_lo + (1 - b) * half
            off = GROUPS[g]["rows"] - p_half
            rs_issue(g, j, send_lo, off, half, dim)
            keep[g] = (pk_lo + b * half, half, off)
            lo[g] = pk_lo + b * half
            length[g] = half

        for g, G in enumerate(GROUPS):
            half = G["rows"] // 2
            dim = G["order"][0]
            b = _keep_high_bit(dim, my)
            send_lo = G["start"] + (1 - b) * half
            dot_rows(send_lo, half)
            rs_issue(g, 0, send_lo, 0, half, dim)
            lo[g] = G["start"] + b * half
            keep[g] = (lo[g], half, 0)
            length[g] = half
        for g, G in enumerate(GROUPS):
            keep_lo, half, _ = keep[g]
            dot_rows(keep_lo, half)

        for j in range(1, nsteps):
            for g in range(NG):
                rs_step(g, j)

        ag_rdma = [None] * NG

        def ag_issue(g, j):
            L = length[g]
            rdma = pltpu.make_async_remote_copy(
                src_ref=zbuf.at[pl.ds(lo[g], L), :],
                dst_ref=zbuf.at[pl.ds(lo[g], L), :],
                send_sem=ag_send.at[g, j],
                recv_sem=ag_recv.at[g, j],
                device_id=(my ^ MASKS[GROUPS[g]["order"][2 - j]],),
                device_id_type=pl.DeviceIdType.MESH,
            )
            rdma.start()
            ag_rdma[g] = rdma

        for g in range(NG):
            rs_rdma[g].wait()
            keep_lo, L, off = keep[g]
            red = (
                zbuf[pl.ds(keep_lo, L), :].astype(F32)
                + rbufs[g][pl.ds(off, L), :].astype(F32)
            )
            silu = red / (1.0 + jnp.exp(-red))
            zbuf[pl.ds(keep_lo, L), :] = silu.astype(BF16)
            ag_issue(g, 0)

        def ag_consume(g, j):
            ag_rdma[g].wait()
            b = _keep_high_bit(GROUPS[g]["order"][2 - j], my)
            lo[g] = lo[g] - b * length[g]
            length[g] = 2 * length[g]
            if j + 1 < nsteps:
                ag_issue(g, j + 1)
            else:
                G = GROUPS[g]
                out_ref[pl.ds(G["start"], G["rows"]), :] = (
                    zbuf[pl.ds(G["start"], G["rows"]), :].astype(F32)
                )

        for j in range(nsteps):
            for g in range(NG):
                ag_consume(g, j)

    return pl.pallas_call(
        body,
        out_shape=jax.ShapeDtypeStruct((m, n), F32),
        in_specs=[
            pl.BlockSpec(memory_space=pltpu.VMEM),
            pl.BlockSpec(memory_space=pltpu.VMEM),
        ],
        out_specs=pl.BlockSpec(memory_space=pltpu.VMEM),
        scratch_shapes=[
            pltpu.VMEM((m, n), BF16),
            pltpu.VMEM((k, n), BF16),
            *[pltpu.VMEM((g["rows"] * 7 // 8, n), BF16) for g in GROUPS],
            pltpu.SemaphoreType.DMA((NG, 3)),
            pltpu.SemaphoreType.DMA((NG, 3)),
            pltpu.SemaphoreType.DMA((NG, 3)),
            pltpu.SemaphoreType.DMA((NG, 3)),
        ],
        compiler_params=pltpu.CompilerParams(
            vmem_limit_bytes=100 * 1024 * 1024,
        ),
    )(A, B)


# device time: 89575 ns/iter; 1.1551x vs baseline; 1.0260x over previous
import jax
import jax.numpy as jnp
from jax import lax
from jax.experimental import pallas as pl
from jax.experimental.pallas import tpu as pltpu

N_DEV = 8
MASKS = {"x": 1, "y": 3, "z": 4}

GROUPS = [
    {"start": 0, "rows": 384, "order": "xyz"},
    {"start": 384, "rows": 384, "order": "yzx"},
    {"start": 768, "rows": 384, "order": "zxy"},
    {"start": 1152, "rows": 384, "order": "xyz"},
    {"start": 1536, "rows": 256, "order": "yzx"},
    {"start": 1792, "rows": 256, "order": "zxy"},
]
NG = len(GROUPS)
F32 = jnp.float32
BF16 = jnp.bfloat16


def _keep_high_bit(dim, my):
    if dim == "x":
        return (my ^ (my >> 1)) & 1
    if dim == "y":
        return (my >> 1) & 1
    return (my >> 2) & 1


def kernel(A, B):
    m, k = A.shape
    _, n = B.shape

    def body(a_ref, b_ref, out_ref, zbuf, bbuf, *scr):
        rbufs = scr[0:NG]
        rs_send, rs_recv, ag_send, ag_recv = scr[NG:NG + 4]
        my = lax.axis_index("i")
        nsteps = 3

        lo = [None] * NG
        length = [g["rows"] for g in GROUPS]
        rs_rdma = [None] * NG
        keep = [None] * NG

        bbuf[...] = b_ref[...].astype(BF16)

        def dot_rows(row_lo, rows):
            zbuf[pl.ds(row_lo, rows), :] = jnp.dot(
                a_ref[pl.ds(row_lo, rows), :].astype(BF16), bbuf[...],
                preferred_element_type=F32,
            ).astype(BF16)

        def rs_issue(g, j, send_lo, buf_off, half, dim):
            rdma = pltpu.make_async_remote_copy(
                src_ref=zbuf.at[pl.ds(send_lo, half), :],
                dst_ref=rbufs[g].at[pl.ds(buf_off, half), :],
                send_sem=rs_send.at[g, j],
                recv_sem=rs_recv.at[g, j],
                device_id=(my ^ MASKS[dim],),
                device_id_type=pl.DeviceIdType.MESH,
            )
            rdma.start()
            rs_rdma[g] = rdma

        def rs_step(g, j):
            rs_rdma[g].wait()
            pk_lo, p_half, p_off = keep[g]
            zbuf[pl.ds(pk_lo, p_half), :] = (
                zbuf[pl.ds(pk_lo, p_half), :]
                + rbufs[g][pl.ds(p_off, p_half), :]
            )
            half = p_half // 2
            dim = GROUPS[g]["order"][j]
            b = _keep_high_bit(dim, my)
            send_lo = pk_lo + (1 - b) * half
            off = GROUPS[g]["rows"] - p_half
            rs_issue(g, j, send_lo, off, half, dim)
            keep[g] = (pk_lo + b * half, half, off)
            lo[g] = pk_lo + b * half
            length[g] = half

        for g, G in enumerate(GROUPS):
            half = G["rows"] // 2
            dim = G["order"][0]
            b = _keep_high_bit(dim, my)
            send_lo = G["start"] + (1 - b) * half
            dot_rows(send_lo, half)
            rs_issue(g, 0, send_lo, 0, half, dim)
            lo[g] = G["start"] + b * half
            keep[g] = (lo[g], half, 0)
            length[g] = half
        for g, G in enumerate(GROUPS):
            keep_lo, half, _ = keep[g]
            dot_rows(keep_lo, half)

        for g in range(NG):
            rs_step(g, 1)

        for g in range(NG):
            rs_rdma[g].wait()
            pk_lo, p_half, p_off = keep[g]
            zbuf[pl.ds(pk_lo, p_half), :] = (
                zbuf[pl.ds(pk_lo, p_half), :]
                + rbufs[g][pl.ds(p_off, p_half), :]
            )
            off2 = GROUPS[g]["rows"] * 3 // 4
            rs_issue(g, 2, pk_lo, off2, p_half, GROUPS[g]["order"][2])
            keep[g] = (pk_lo, p_half, off2)

        ag_rdma = [None] * NG

        def ag_issue(g, j):
            L = length[g]
            rdma = pltpu.make_async_remote_copy(
                src_ref=zbuf.at[pl.ds(lo[g], L), :],
                dst_ref=zbuf.at[pl.ds(lo[g], L), :],
                send_sem=ag_send.at[g, j],
                recv_sem=ag_recv.at[g, j],
                device_id=(my ^ MASKS[GROUPS[g]["order"][1 - j]],),
                device_id_type=pl.DeviceIdType.MESH,
            )
            rdma.start()
            ag_rdma[g] = rdma

        for g in range(NG):
            rs_rdma[g].wait()
            keep_lo, L, off = keep[g]
            red = (
                zbuf[pl.ds(keep_lo, L), :].astype(F32)
                + rbufs[g][pl.ds(off, L), :].astype(F32)
            )
            silu = red / (1.0 + jnp.exp(-red))
            zbuf[pl.ds(keep_lo, L), :] = silu.astype(BF16)
            ag_issue(g, 0)

        def ag_consume(g, j):
            ag_rdma[g].wait()
            b = _keep_high_bit(GROUPS[g]["order"][1 - j], my)
            lo[g] = lo[g] - b * length[g]
            length[g] = 2 * length[g]
            if j + 1 < 2:
                ag_issue(g, j + 1)
            else:
                G = GROUPS[g]
                out_ref[pl.ds(G["start"], G["rows"]), :] = (
                    zbuf[pl.ds(G["start"], G["rows"]), :].astype(F32)
                )

        for j in range(2):
            for g in range(NG):
                ag_consume(g, j)

    return pl.pallas_call(
        body,
        out_shape=jax.ShapeDtypeStruct((m, n), F32),
        in_specs=[
            pl.BlockSpec(memory_space=pltpu.VMEM),
            pl.BlockSpec(memory_space=pltpu.VMEM),
        ],
        out_specs=pl.BlockSpec(memory_space=pltpu.VMEM),
        scratch_shapes=[
            pltpu.VMEM((m, n), BF16),
            pltpu.VMEM((k, n), BF16),
            *[pltpu.VMEM((g["rows"], n), BF16) for g in GROUPS],
            pltpu.SemaphoreType.DMA((NG, 3)),
            pltpu.SemaphoreType.DMA((NG, 3)),
            pltpu.SemaphoreType.DMA((NG, 2)),
            pltpu.SemaphoreType.DMA((NG, 2)),
        ],
        compiler_params=pltpu.CompilerParams(
            vmem_limit_bytes=100 * 1024 * 1024,
        ),
    )(A, B)


# device time: 86877 ns/iter; 1.1910x vs baseline; 1.0311x over previous
import jax
import jax.numpy as jnp
from jax import lax
from jax.experimental import pallas as pl
from jax.experimental.pallas import tpu as pltpu

N_DEV = 8
MASKS = {"x": 1, "y": 3, "z": 4}

GROUPS = [
    {"start": 0, "rows": 384, "order": "xyz"},
    {"start": 384, "rows": 384, "order": "yzx"},
    {"start": 768, "rows": 384, "order": "zxy"},
    {"start": 1152, "rows": 384, "order": "xyz"},
    {"start": 1536, "rows": 256, "order": "yzx"},
    {"start": 1792, "rows": 256, "order": "zxy"},
]
NG = len(GROUPS)
F32 = jnp.float32
BF16 = jnp.bfloat16


def _keep_high_bit(dim, my):
    if dim == "x":
        return (my ^ (my >> 1)) & 1
    if dim == "y":
        return (my >> 1) & 1
    return (my >> 2) & 1


def kernel(A, B):
    m, k = A.shape
    _, n = B.shape

    def body(a_ref, b_ref, out_ref, zbuf, bbuf, fbuf, *scr):
        rbufs = scr[0:NG]
        rs_send, rs_recv, ag_send, ag_recv, out_sem = scr[NG:NG + 5]
        my = lax.axis_index("i")
        nsteps = 3

        lo = [None] * NG
        length = [g["rows"] for g in GROUPS]
        rs_rdma = [None] * NG
        keep = [None] * NG

        bbuf[...] = b_ref[...].astype(BF16)

        def dot_rows(row_lo, rows):
            zbuf[pl.ds(row_lo, rows), :] = jnp.dot(
                a_ref[pl.ds(row_lo, rows), :].astype(BF16), bbuf[...],
                preferred_element_type=F32,
            ).astype(BF16)

        def rs_issue(g, j, send_lo, buf_off, half, dim):
            rdma = pltpu.make_async_remote_copy(
                src_ref=zbuf.at[pl.ds(send_lo, half), :],
                dst_ref=rbufs[g].at[pl.ds(buf_off, half), :],
                send_sem=rs_send.at[g, j],
                recv_sem=rs_recv.at[g, j],
                device_id=(my ^ MASKS[dim],),
                device_id_type=pl.DeviceIdType.MESH,
            )
            rdma.start()
            rs_rdma[g] = rdma

        def rs_step(g, j):
            rs_rdma[g].wait()
            pk_lo, p_half, p_off = keep[g]
            zbuf[pl.ds(pk_lo, p_half), :] = (
                zbuf[pl.ds(pk_lo, p_half), :]
                + rbufs[g][pl.ds(p_off, p_half), :]
            )
            half = p_half // 2
            dim = GROUPS[g]["order"][j]
            b = _keep_high_bit(dim, my)
            send_lo = pk_lo + (1 - b) * half
            off = GROUPS[g]["rows"] - p_half
            rs_issue(g, j, send_lo, off, half, dim)
            keep[g] = (pk_lo + b * half, half, off)
            lo[g] = pk_lo + b * half
            length[g] = half

        for g, G in enumerate(GROUPS):
            half = G["rows"] // 2
            dim = G["order"][0]
            b = _keep_high_bit(dim, my)
            send_lo = G["start"] + (1 - b) * half
            dot_rows(send_lo, half)
            rs_issue(g, 0, send_lo, 0, half, dim)
            lo[g] = G["start"] + b * half
            keep[g] = (lo[g], half, 0)
            length[g] = half
        for g, G in enumerate(GROUPS):
            keep_lo, half, _ = keep[g]
            dot_rows(keep_lo, half)

        for g in range(NG):
            rs_step(g, 1)

        for g in range(NG):
            rs_rdma[g].wait()
            pk_lo, p_half, p_off = keep[g]
            zbuf[pl.ds(pk_lo, p_half), :] = (
                zbuf[pl.ds(pk_lo, p_half), :]
                + rbufs[g][pl.ds(p_off, p_half), :]
            )
            off2 = GROUPS[g]["rows"] * 3 // 4
            rs_issue(g, 2, pk_lo, off2, p_half, GROUPS[g]["order"][2])
            keep[g] = (pk_lo, p_half, off2)

        ag_rdma = [None] * NG

        def ag_issue(g, j):
            L = length[g]
            rdma = pltpu.make_async_remote_copy(
                src_ref=zbuf.at[pl.ds(lo[g], L), :],
                dst_ref=zbuf.at[pl.ds(lo[g], L), :],
                send_sem=ag_send.at[g, j],
                recv_sem=ag_recv.at[g, j],
                device_id=(my ^ MASKS[GROUPS[g]["order"][1 - j]],),
                device_id_type=pl.DeviceIdType.MESH,
            )
            rdma.start()
            ag_rdma[g] = rdma

        for g in range(NG):
            rs_rdma[g].wait()
            keep_lo, L, off = keep[g]
            red = (
                zbuf[pl.ds(keep_lo, L), :].astype(F32)
                + rbufs[g][pl.ds(off, L), :].astype(F32)
            )
            silu = red / (1.0 + jnp.exp(-red))
            zbuf[pl.ds(keep_lo, L), :] = silu.astype(BF16)
            ag_issue(g, 0)

        def ag_consume(g, j):
            ag_rdma[g].wait()
            b = _keep_high_bit(GROUPS[g]["order"][1 - j], my)
            lo[g] = lo[g] - b * length[g]
            length[g] = 2 * length[g]
            if j + 1 < 2:
                ag_issue(g, j + 1)
            else:
                G = GROUPS[g]
                fbuf[pl.ds(G["start"], G["rows"]), :] = (
                    zbuf[pl.ds(G["start"], G["rows"]), :].astype(F32)
                )
                cp = pltpu.make_async_copy(
                    fbuf.at[pl.ds(G["start"], G["rows"]), :],
                    out_ref.at[pl.ds(G["start"], G["rows"]), :],
                    out_sem.at[g],
                )
                cp.start()
                out_cp[g] = cp

        out_cp = [None] * NG
        for j in range(2):
            for g in range(NG):
                ag_consume(g, j)
        for g in range(NG):
            out_cp[g].wait()

    return pl.pallas_call(
        body,
        out_shape=jax.ShapeDtypeStruct((m, n), F32),
        in_specs=[
            pl.BlockSpec(memory_space=pltpu.VMEM),
            pl.BlockSpec(memory_space=pltpu.VMEM),
        ],
        out_specs=pl.BlockSpec(memory_space=pl.ANY),
        scratch_shapes=[
            pltpu.VMEM((m, n), BF16),
            pltpu.VMEM((k, n), BF16),
            pltpu.VMEM((m, n), F32),
            *[pltpu.VMEM((g["rows"], n), BF16) for g in GROUPS],
            pltpu.SemaphoreType.DMA((NG, 3)),
            pltpu.SemaphoreType.DMA((NG, 3)),
            pltpu.SemaphoreType.DMA((NG, 2)),
            pltpu.SemaphoreType.DMA((NG, 2)),
            pltpu.SemaphoreType.DMA((NG,)),
        ],
        compiler_params=pltpu.CompilerParams(
            vmem_limit_bytes=100 * 1024 * 1024,
        ),
    )(A, B)
